# baseline (speedup 1.0000x reference)
import functools

import jax
import jax.numpy as jnp
from jax import lax
from jax.experimental import pallas as pl
from jax.experimental.pallas import tpu as pltpu
from jax.experimental.pallas import tpu_sc as plsc

N = 10000
E = 320000
D_IN = 165
D_HID = 128

NC = 2
NS = 16
NW = NC * NS
CH = 128
NCHUNK = 80
EPW = NCHUNK * CH
E2 = NW * EPW
NBUF = 4

NPAD = 10240
RPT = NPAD // NS

_mesh = plsc.VectorSubcoreMesh(core_axis_name="c", subcore_axis_name="s")


def _zeros16():
    return jnp.zeros((16,), jnp.float32)


def _sc_degree(dstp):
    @functools.partial(
        pl.kernel,
        mesh=_mesh,
        out_type=(
            jax.ShapeDtypeStruct((NPAD,), jnp.float32),
            jax.ShapeDtypeStruct((NPAD,), jnp.float32),
        ),
        scratch_types=[
            [pltpu.VMEM((CH,), jnp.int32)] * NBUF,
            [pltpu.SemaphoreType.DMA] * NBUF,
            pltpu.VMEM((CH,), jnp.float32),
            pltpu.VMEM((RPT,), jnp.float32),
            pltpu.VMEM_SHARED((NPAD,), jnp.float32),
        ],
    )
    def deg_kernel(dst_hbm, out_a, out_b, didx, sems, ones_v, zbuf, acc):
        cid = lax.axis_index("c")
        sid = lax.axis_index("s")
        base = (cid * NS + sid) * EPW
        for i in range(CH // 16):
            ones_v[pl.ds(i * 16, 16)] = jnp.ones((16,), jnp.float32)
        for i in range(RPT // 16):
            zbuf[pl.ds(i * 16, 16)] = _zeros16()
        pltpu.sync_copy(zbuf, acc.at[pl.ds(sid * RPT, RPT)])
        plsc.subcore_barrier()

        def body(g, carry):
            ld = []
            for b in range(NBUF):
                j = g * NBUF + b
                ld.append(pltpu.async_copy(
                    dst_hbm.at[pl.ds(base + j * CH, CH)], didx[b], sems[b]))
            sd = []
            for b in range(NBUF):
                ld[b].wait()
                sd.append(pltpu.async_copy(
                    ones_v, acc.at[didx[b]], sems[b], add=True))
            for b in range(NBUF):
                sd[b].wait()
            return carry

        lax.fori_loop(0, NCHUNK // NBUF, body, 0)
        plsc.subcore_barrier()

        @pl.when(cid == 0)
        def _():
            pltpu.sync_copy(acc.at[pl.ds(sid * RPT, RPT)],
                            out_a.at[pl.ds(sid * RPT, RPT)])

        @pl.when(cid == 1)
        def _():
            pltpu.sync_copy(acc.at[pl.ds(sid * RPT, RPT)],
                            out_b.at[pl.ds(sid * RPT, RPT)])

    return deg_kernel(dstp)


CG = 80
NCG = EPW // CG
NCG0 = 64
NCG1 = 192
E0 = NS * NCG0 * CG


def _sc_agg_rows(hs, srcp, dstp):
    @functools.partial(
        pl.kernel,
        mesh=_mesh,
        out_type=(
            jax.ShapeDtypeStruct((NPAD, D_HID), jnp.float32),
            jax.ShapeDtypeStruct((NPAD, D_HID), jnp.float32),
        ),
        scratch_types=[
            [pltpu.VMEM((CG,), jnp.int32)] * 4,
            [pltpu.VMEM((CG,), jnp.int32)] * 4,
            [pltpu.VMEM((CG, D_HID), jnp.float32)] * 2,
            [pltpu.SemaphoreType.DMA] * 4,
            [pltpu.SemaphoreType.DMA] * 2,
            pltpu.SemaphoreType.DMA,
            pltpu.VMEM_SHARED((NPAD, D_HID), jnp.float32),
        ],
    )
    def agg_kernel(hs_hbm, src_hbm, dst_hbm, out_a, out_b,
                   sidx, didx, rows, sem_i, sem_s, sem_g, acc):
        cid = lax.axis_index("c")
        sid = lax.axis_index("s")
        base = (cid * NS + sid) * EPW

        def _wait_idx(b):
            pltpu.make_async_copy(
                src_hbm.at[pl.ds(0, CG)], sidx[b], sem_i[b]).wait()
            pltpu.make_async_copy(
                src_hbm.at[pl.ds(0, CG)], didx[b], sem_i[b]).wait()

        def _wait_scat(r):
            pltpu.make_async_copy(
                hs_hbm.at[pl.ds(0, CG)], rows[r], sem_s[r]).wait()

        def _load_idx(base, j, b):
            pltpu.async_copy(
                src_hbm.at[pl.ds(base + j * CG, CG)], sidx[b], sem_i[b])
            pltpu.async_copy(
                dst_hbm.at[pl.ds(base + j * CG, CG)], didx[b], sem_i[b])

        def zb(i, carry):
            rows[0][i // 8, pl.ds((i % 8) * 16, 16)] = _zeros16()
            return carry

        lax.fori_loop(0, (CG * D_HID) // 16, zb, 0)

        def zc(k, carry):
            pltpu.sync_copy(rows[0], acc.at[pl.ds(sid * RPT + k * CG, CG)])
            return carry

        lax.fori_loop(0, RPT // CG, zc, 0)
        plsc.subcore_barrier()

        def run_range(base, nchunks):
            def body(j, carry):
                _load_idx(base, j, 0)
                _wait_idx(0)
                pltpu.async_copy(hs_hbm.at[sidx[0]], rows[0], sem_g).wait()
                pltpu.sync_copy(rows[0], acc.at[didx[0]], add=True)
                return carry

            lax.fori_loop(0, nchunks, body, 0)

        @pl.when(cid == 0)
        def _():
            run_range(sid * (NCG0 * CG), NCG0)

        @pl.when(cid == 1)
        def _():
            run_range(E0 + sid * (NCG1 * CG), NCG1)
        plsc.subcore_barrier()

        @pl.when(cid == 0)
        def _():
            pltpu.sync_copy(acc.at[pl.ds(sid * RPT, RPT)],
                            out_a.at[pl.ds(sid * RPT, RPT)])

        @pl.when(cid == 1)
        def _():
            pltpu.sync_copy(acc.at[pl.ds(sid * RPT, RPT)],
                            out_b.at[pl.ds(sid * RPT, RPT)])

    return agg_kernel(hs, srcp, dstp)


def _sc_agg_scalar(vs, srcp, dstp):
    @functools.partial(
        pl.kernel,
        mesh=_mesh,
        out_type=(
            jax.ShapeDtypeStruct((NPAD,), jnp.float32),
            jax.ShapeDtypeStruct((NPAD,), jnp.float32),
        ),
        scratch_types=[
            [pltpu.VMEM((CH,), jnp.int32)] * NBUF,
            [pltpu.VMEM((CH,), jnp.int32)] * NBUF,
            [pltpu.VMEM((CH,), jnp.float32)] * NBUF,
            [pltpu.SemaphoreType.DMA] * NBUF,
            pltpu.VMEM((RPT,), jnp.float32),
            pltpu.VMEM_SHARED((NPAD,), jnp.float32),
        ],
    )
    def aggs_kernel(vs_hbm, src_hbm, dst_hbm, out_a, out_b,
                    sidx, didx, vals, sems, zbuf, acc):
        cid = lax.axis_index("c")
        sid = lax.axis_index("s")
        base = (cid * NS + sid) * EPW
        for i in range(RPT // 16):
            zbuf[pl.ds(i * 16, 16)] = _zeros16()
        pltpu.sync_copy(zbuf, acc.at[pl.ds(sid * RPT, RPT)])
        plsc.subcore_barrier()

        def body(g, carry):
            ld = []
            for b in range(NBUF):
                j = g * NBUF + b
                d1 = pltpu.async_copy(
                    src_hbm.at[pl.ds(base + j * CH, CH)], sidx[b], sems[b])
                d2 = pltpu.async_copy(
                    dst_hbm.at[pl.ds(base + j * CH, CH)], didx[b], sems[b])
                ld.append((d1, d2))
            gd = []
            for b in range(NBUF):
                ld[b][0].wait()
                ld[b][1].wait()
                gd.append(pltpu.async_copy(
                    vs_hbm.at[sidx[b]], vals[b], sems[b]))
            sd = []
            for b in range(NBUF):
                gd[b].wait()
                sd.append(pltpu.async_copy(
                    vals[b], acc.at[didx[b]], sems[b], add=True))
            for b in range(NBUF):
                sd[b].wait()
            return carry

        lax.fori_loop(0, NCHUNK // NBUF, body, 0)
        plsc.subcore_barrier()

        @pl.when(cid == 0)
        def _():
            pltpu.sync_copy(acc.at[pl.ds(sid * RPT, RPT)],
                            out_a.at[pl.ds(sid * RPT, RPT)])

        @pl.when(cid == 1)
        def _():
            pltpu.sync_copy(acc.at[pl.ds(sid * RPT, RPT)],
                            out_b.at[pl.ds(sid * RPT, RPT)])

    return aggs_kernel(vs, srcp, dstp)


BN = 1000


def _tc_mm1_body(x_ref, w_ref, da_ref, db_ref, hs_ref):
    dinv = lax.rsqrt(da_ref[...] + db_ref[...] + 1.0)
    h = jnp.dot(x_ref[...], w_ref[...], preferred_element_type=jnp.float32)
    hs_ref[...] = h * dinv


def _tc_mm1(x, w1, da, db):
    return pl.pallas_call(
        _tc_mm1_body,
        grid=(N // BN,),
        in_specs=[
            pl.BlockSpec((BN, D_IN), lambda i: (i, 0)),
            pl.BlockSpec((D_IN, D_HID), lambda i: (0, 0)),
            pl.BlockSpec((BN, 1), lambda i: (i, 0)),
            pl.BlockSpec((BN, 1), lambda i: (i, 0)),
        ],
        out_specs=pl.BlockSpec((BN, D_HID), lambda i: (i, 0)),
        out_shape=jax.ShapeDtypeStruct((NPAD, D_HID), jnp.float32),
    )(x, w1, da, db)


def _tc_mm2_body(aa_ref, ab_ref, hs_ref, da_ref, db_ref, b1_ref, w2t_ref,
                 vs_ref):
    dinv = lax.rsqrt(da_ref[...] + db_ref[...] + 1.0)
    pre = dinv * (aa_ref[...] + ab_ref[...] + hs_ref[...]) + b1_ref[...]
    h = jnp.maximum(pre, 0.0)
    z = jnp.sum(h * w2t_ref[...], axis=1, keepdims=True)
    vs_ref[...] = dinv * z


def _tc_mm2(aa, ab, hs, da, db, b1r, w2t):
    return pl.pallas_call(
        _tc_mm2_body,
        grid=(N // BN,),
        in_specs=[
            pl.BlockSpec((BN, D_HID), lambda i: (i, 0)),
            pl.BlockSpec((BN, D_HID), lambda i: (i, 0)),
            pl.BlockSpec((BN, D_HID), lambda i: (i, 0)),
            pl.BlockSpec((BN, 1), lambda i: (i, 0)),
            pl.BlockSpec((BN, 1), lambda i: (i, 0)),
            pl.BlockSpec((1, D_HID), lambda i: (0, 0)),
            pl.BlockSpec((1, D_HID), lambda i: (0, 0)),
        ],
        out_specs=pl.BlockSpec((BN, 1), lambda i: (i, 0)),
        out_shape=jax.ShapeDtypeStruct((NPAD, 1), jnp.float32),
    )(aa, ab, hs, da, db, b1r, w2t)


def _tc_fin_body(va_ref, vb_ref, vs_ref, da_ref, db_ref, b2_ref, out_ref):
    dinv = lax.rsqrt(da_ref[...] + db_ref[...] + 1.0)
    z = dinv * (va_ref[...] + vb_ref[...] + vs_ref[...]) + b2_ref[0, 0]
    out_ref[...] = jax.nn.sigmoid(z)


def _tc_fin(va, vb, vs, da, db, b2):
    shp = (80, 125)
    args = [a.reshape(shp) for a in (va, vb, vs, da, db)]
    out = pl.pallas_call(
        _tc_fin_body,
        in_specs=[pl.BlockSpec(shp, lambda: (0, 0))] * 5
        + [pl.BlockSpec((1, 1), lambda: (0, 0))],
        out_specs=pl.BlockSpec(shp, lambda: (0, 0)),
        out_shape=jax.ShapeDtypeStruct(shp, jnp.float32),
    )(*args, b2.reshape(1, 1))
    return out.reshape(N, 1)


def kernel(x, edge_index, W1, b1, W2, b2):
    src = edge_index[0]
    dst = edge_index[1]
    pad = jnp.full((E2 - E,), N, jnp.int32)
    srcp = jnp.concatenate([src, pad])
    dstp = jnp.concatenate([dst, pad])

    deg_a, deg_b = _sc_degree(dstp)
    da = deg_a[:N].reshape(N, 1)
    db = deg_b[:N].reshape(N, 1)

    hs = _tc_mm1(x, W1, da, db)
    acc_a, acc_b = _sc_agg_rows(hs, srcp, dstp)

    vs = _tc_mm2(acc_a, acc_b, hs, da, db,
                 b1.reshape(1, D_HID), W2.reshape(1, D_HID))
    vsf = vs.reshape(NPAD)

    va, vb = _sc_agg_scalar(vsf, srcp, dstp)
    out = _tc_fin(va[:N], vb[:N], vsf[:N], da.reshape(N), db.reshape(N), b2)
    return out


# speedup vs baseline: 1.3727x; 1.3727x over previous
import functools

import jax
import jax.numpy as jnp
from jax import lax
from jax.experimental import pallas as pl
from jax.experimental.pallas import tpu as pltpu
from jax.experimental.pallas import tpu_sc as plsc

N = 10000
E = 320000
D_IN = 165
D_HID = 128

NC = 2
NS = 16
NW = NC * NS
CH = 128
NCHUNK = 80
EPW = NCHUNK * CH
E2 = NW * EPW
NBUF = 4

NPAD = 10240
RPT = NPAD // NS

_mesh = plsc.VectorSubcoreMesh(core_axis_name="c", subcore_axis_name="s")


def _zeros16():
    return jnp.zeros((16,), jnp.float32)


def _sc_degree(dstp):
    @functools.partial(
        pl.kernel,
        mesh=_mesh,
        out_type=(
            jax.ShapeDtypeStruct((NPAD,), jnp.float32),
            jax.ShapeDtypeStruct((NPAD,), jnp.float32),
        ),
        scratch_types=[
            [pltpu.VMEM((CH,), jnp.int32)] * NBUF,
            [pltpu.SemaphoreType.DMA] * NBUF,
            pltpu.VMEM((CH,), jnp.float32),
            pltpu.VMEM((RPT,), jnp.float32),
            pltpu.VMEM_SHARED((NPAD,), jnp.float32),
        ],
    )
    def deg_kernel(dst_hbm, out_a, out_b, didx, sems, ones_v, zbuf, acc):
        cid = lax.axis_index("c")
        sid = lax.axis_index("s")
        base = (cid * NS + sid) * EPW
        for i in range(CH // 16):
            ones_v[pl.ds(i * 16, 16)] = jnp.ones((16,), jnp.float32)
        for i in range(RPT // 16):
            zbuf[pl.ds(i * 16, 16)] = _zeros16()
        pltpu.sync_copy(zbuf, acc.at[pl.ds(sid * RPT, RPT)])
        plsc.subcore_barrier()

        def body(g, carry):
            ld = []
            for b in range(NBUF):
                j = g * NBUF + b
                ld.append(pltpu.async_copy(
                    dst_hbm.at[pl.ds(base + j * CH, CH)], didx[b], sems[b]))
            sd = []
            for b in range(NBUF):
                ld[b].wait()
                sd.append(pltpu.async_copy(
                    ones_v, acc.at[didx[b]], sems[b], add=True))
            for b in range(NBUF):
                sd[b].wait()
            return carry

        lax.fori_loop(0, NCHUNK // NBUF, body, 0)
        plsc.subcore_barrier()

        @pl.when(cid == 0)
        def _():
            pltpu.sync_copy(acc.at[pl.ds(sid * RPT, RPT)],
                            out_a.at[pl.ds(sid * RPT, RPT)])

        @pl.when(cid == 1)
        def _():
            pltpu.sync_copy(acc.at[pl.ds(sid * RPT, RPT)],
                            out_b.at[pl.ds(sid * RPT, RPT)])

    return deg_kernel(dstp)


CG = 80
NCG = EPW // CG
NCG0 = 192
NCG1 = 64
E0 = NS * NCG0 * CG


def _sc_agg_rows(hs, srcp, dstp):
    @functools.partial(
        pl.kernel,
        mesh=_mesh,
        out_type=(
            jax.ShapeDtypeStruct((NPAD, D_HID), jnp.float32),
            jax.ShapeDtypeStruct((NPAD, D_HID), jnp.float32),
        ),
        scratch_types=[
            [pltpu.VMEM((CG,), jnp.int32)] * 4,
            [pltpu.VMEM((CG,), jnp.int32)] * 4,
            [pltpu.VMEM((CG, D_HID), jnp.float32)] * 2,
            [pltpu.SemaphoreType.DMA] * 4,
            [pltpu.SemaphoreType.DMA] * 2,
            pltpu.SemaphoreType.DMA,
            pltpu.VMEM_SHARED((NPAD, D_HID), jnp.float32),
        ],
    )
    def agg_kernel(hs_hbm, src_hbm, dst_hbm, out_a, out_b,
                   sidx, didx, rows, sem_i, sem_s, sem_g, acc):
        cid = lax.axis_index("c")
        sid = lax.axis_index("s")
        base = (cid * NS + sid) * EPW

        def _wait_idx(b):
            pltpu.make_async_copy(
                src_hbm.at[pl.ds(0, CG)], sidx[b], sem_i[b]).wait()
            pltpu.make_async_copy(
                src_hbm.at[pl.ds(0, CG)], didx[b], sem_i[b]).wait()

        def _wait_scat(r):
            pltpu.make_async_copy(
                hs_hbm.at[pl.ds(0, CG)], rows[r], sem_s[r]).wait()

        def _load_idx(base, j, b):
            pltpu.async_copy(
                src_hbm.at[pl.ds(base + j * CG, CG)], sidx[b], sem_i[b])
            pltpu.async_copy(
                dst_hbm.at[pl.ds(base + j * CG, CG)], didx[b], sem_i[b])

        def zb(i, carry):
            rows[0][i // 8, pl.ds((i % 8) * 16, 16)] = _zeros16()
            return carry

        lax.fori_loop(0, (CG * D_HID) // 16, zb, 0)

        def zc(k, carry):
            pltpu.sync_copy(rows[0], acc.at[pl.ds(sid * RPT + k * CG, CG)])
            return carry

        lax.fori_loop(0, RPT // CG, zc, 0)
        plsc.subcore_barrier()

        def run_range(base, nchunks):
            def body(j, carry):
                _load_idx(base, j, 0)
                _wait_idx(0)
                pltpu.async_copy(hs_hbm.at[sidx[0]], rows[0], sem_g).wait()
                pltpu.sync_copy(rows[0], acc.at[didx[0]], add=True)
                return carry

            lax.fori_loop(0, nchunks, body, 0)

        @pl.when(cid == 0)
        def _():
            run_range(sid * (NCG0 * CG), NCG0)

        @pl.when(cid == 1)
        def _():
            run_range(E0 + sid * (NCG1 * CG), NCG1)
        plsc.subcore_barrier()

        @pl.when(cid == 0)
        def _():
            pltpu.sync_copy(acc.at[pl.ds(sid * RPT, RPT)],
                            out_a.at[pl.ds(sid * RPT, RPT)])

        @pl.when(cid == 1)
        def _():
            pltpu.sync_copy(acc.at[pl.ds(sid * RPT, RPT)],
                            out_b.at[pl.ds(sid * RPT, RPT)])

    return agg_kernel(hs, srcp, dstp)


def _sc_agg_scalar(vs, srcp, dstp):
    @functools.partial(
        pl.kernel,
        mesh=_mesh,
        out_type=(
            jax.ShapeDtypeStruct((NPAD,), jnp.float32),
            jax.ShapeDtypeStruct((NPAD,), jnp.float32),
        ),
        scratch_types=[
            [pltpu.VMEM((CH,), jnp.int32)] * NBUF,
            [pltpu.VMEM((CH,), jnp.int32)] * NBUF,
            [pltpu.VMEM((CH,), jnp.float32)] * NBUF,
            [pltpu.SemaphoreType.DMA] * NBUF,
            pltpu.VMEM((RPT,), jnp.float32),
            pltpu.VMEM_SHARED((NPAD,), jnp.float32),
        ],
    )
    def aggs_kernel(vs_hbm, src_hbm, dst_hbm, out_a, out_b,
                    sidx, didx, vals, sems, zbuf, acc):
        cid = lax.axis_index("c")
        sid = lax.axis_index("s")
        base = (cid * NS + sid) * EPW
        for i in range(RPT // 16):
            zbuf[pl.ds(i * 16, 16)] = _zeros16()
        pltpu.sync_copy(zbuf, acc.at[pl.ds(sid * RPT, RPT)])
        plsc.subcore_barrier()

        def body(g, carry):
            ld = []
            for b in range(NBUF):
                j = g * NBUF + b
                d1 = pltpu.async_copy(
                    src_hbm.at[pl.ds(base + j * CH, CH)], sidx[b], sems[b])
                d2 = pltpu.async_copy(
                    dst_hbm.at[pl.ds(base + j * CH, CH)], didx[b], sems[b])
                ld.append((d1, d2))
            gd = []
            for b in range(NBUF):
                ld[b][0].wait()
                ld[b][1].wait()
                gd.append(pltpu.async_copy(
                    vs_hbm.at[sidx[b]], vals[b], sems[b]))
            sd = []
            for b in range(NBUF):
                gd[b].wait()
                sd.append(pltpu.async_copy(
                    vals[b], acc.at[didx[b]], sems[b], add=True))
            for b in range(NBUF):
                sd[b].wait()
            return carry

        lax.fori_loop(0, NCHUNK // NBUF, body, 0)
        plsc.subcore_barrier()

        @pl.when(cid == 0)
        def _():
            pltpu.sync_copy(acc.at[pl.ds(sid * RPT, RPT)],
                            out_a.at[pl.ds(sid * RPT, RPT)])

        @pl.when(cid == 1)
        def _():
            pltpu.sync_copy(acc.at[pl.ds(sid * RPT, RPT)],
                            out_b.at[pl.ds(sid * RPT, RPT)])

    return aggs_kernel(vs, srcp, dstp)


BN = 1000


def _tc_mm1_body(x_ref, w_ref, da_ref, db_ref, hs_ref):
    dinv = lax.rsqrt(da_ref[...] + db_ref[...] + 1.0)
    h = jnp.dot(x_ref[...], w_ref[...], preferred_element_type=jnp.float32)
    hs_ref[...] = h * dinv


def _tc_mm1(x, w1, da, db):
    return pl.pallas_call(
        _tc_mm1_body,
        grid=(N // BN,),
        in_specs=[
            pl.BlockSpec((BN, D_IN), lambda i: (i, 0)),
            pl.BlockSpec((D_IN, D_HID), lambda i: (0, 0)),
            pl.BlockSpec((BN, 1), lambda i: (i, 0)),
            pl.BlockSpec((BN, 1), lambda i: (i, 0)),
        ],
        out_specs=pl.BlockSpec((BN, D_HID), lambda i: (i, 0)),
        out_shape=jax.ShapeDtypeStruct((NPAD, D_HID), jnp.float32),
    )(x, w1, da, db)


def _tc_mm2_body(aa_ref, ab_ref, hs_ref, da_ref, db_ref, b1_ref, w2t_ref,
                 vs_ref):
    dinv = lax.rsqrt(da_ref[...] + db_ref[...] + 1.0)
    pre = dinv * (aa_ref[...] + ab_ref[...] + hs_ref[...]) + b1_ref[...]
    h = jnp.maximum(pre, 0.0)
    z = jnp.sum(h * w2t_ref[...], axis=1, keepdims=True)
    vs_ref[...] = dinv * z


def _tc_mm2(aa, ab, hs, da, db, b1r, w2t):
    return pl.pallas_call(
        _tc_mm2_body,
        grid=(N // BN,),
        in_specs=[
            pl.BlockSpec((BN, D_HID), lambda i: (i, 0)),
            pl.BlockSpec((BN, D_HID), lambda i: (i, 0)),
            pl.BlockSpec((BN, D_HID), lambda i: (i, 0)),
            pl.BlockSpec((BN, 1), lambda i: (i, 0)),
            pl.BlockSpec((BN, 1), lambda i: (i, 0)),
            pl.BlockSpec((1, D_HID), lambda i: (0, 0)),
            pl.BlockSpec((1, D_HID), lambda i: (0, 0)),
        ],
        out_specs=pl.BlockSpec((BN, 1), lambda i: (i, 0)),
        out_shape=jax.ShapeDtypeStruct((NPAD, 1), jnp.float32),
    )(aa, ab, hs, da, db, b1r, w2t)


def _tc_fin_body(va_ref, vb_ref, vs_ref, da_ref, db_ref, b2_ref, out_ref):
    dinv = lax.rsqrt(da_ref[...] + db_ref[...] + 1.0)
    z = dinv * (va_ref[...] + vb_ref[...] + vs_ref[...]) + b2_ref[0, 0]
    out_ref[...] = jax.nn.sigmoid(z)


def _tc_fin(va, vb, vs, da, db, b2):
    shp = (80, 125)
    args = [a.reshape(shp) for a in (va, vb, vs, da, db)]
    out = pl.pallas_call(
        _tc_fin_body,
        in_specs=[pl.BlockSpec(shp, lambda: (0, 0))] * 5
        + [pl.BlockSpec((1, 1), lambda: (0, 0))],
        out_specs=pl.BlockSpec(shp, lambda: (0, 0)),
        out_shape=jax.ShapeDtypeStruct(shp, jnp.float32),
    )(*args, b2.reshape(1, 1))
    return out.reshape(N, 1)


def kernel(x, edge_index, W1, b1, W2, b2):
    src = edge_index[0]
    dst = edge_index[1]
    pad = jnp.full((E2 - E,), N, jnp.int32)
    srcp = jnp.concatenate([src, pad])
    dstp = jnp.concatenate([dst, pad])

    deg_a, deg_b = _sc_degree(dstp)
    da = deg_a[:N].reshape(N, 1)
    db = deg_b[:N].reshape(N, 1)

    hs = _tc_mm1(x, W1, da, db)
    acc_a, acc_b = _sc_agg_rows(hs, srcp, dstp)

    vs = _tc_mm2(acc_a, acc_b, hs, da, db,
                 b1.reshape(1, D_HID), W2.reshape(1, D_HID))
    vsf = vs.reshape(NPAD)

    va, vb = _sc_agg_scalar(vsf, srcp, dstp)
    out = _tc_fin(va[:N], vb[:N], vsf[:N], da.reshape(N), db.reshape(N), b2)
    return out
